# Initial kernel scaffold; baseline (speedup 1.0000x reference)
import jax
import jax.numpy as jnp
from jax.experimental import pallas as pl


def kernel(inputs, codebook):
    raise NotImplementedError("write your pallas kernel here")


# speedup vs baseline: 2.1976x; 2.1976x over previous
import functools

import jax
import jax.numpy as jnp
from jax import lax
from jax.experimental import pallas as pl
from jax.experimental.pallas import tpu as pltpu
from jax.experimental.pallas import tpu_sc as plsc

N = 8192
D = 64
K = 1024
BLK = 1024
GRID = N // BLK

_EMB_W = 1.0
_COMMIT_W = 0.25
_ENTROPY_W = 0.1


def _tc_body(x_ref, cbt_ref, tok_ref, loss_ref, c2_s, colmin_s, active_s, rowsum_s):
    i = pl.program_id(0)

    @pl.when(i == 0)
    def _init():
        cbt = cbt_ref[...]
        c2_s[...] = jnp.sum(cbt * cbt, axis=0, keepdims=True)
        colmin_s[...] = jnp.full((1, K), jnp.inf, jnp.float32)
        active_s[...] = jnp.zeros((1, K), jnp.float32)
        rowsum_s[0, 0] = 0.0

    x = x_ref[...]
    v2 = jnp.sum(x * x, axis=1, keepdims=True)
    cross = jnp.dot(x.astype(jnp.bfloat16), cbt_ref[...].astype(jnp.bfloat16),
                    preferred_element_type=jnp.float32)
    dist = (v2 - 2.0 * cross) + c2_s[...]

    rowmin = jnp.min(dist, axis=1, keepdims=True)
    iota = lax.broadcasted_iota(jnp.int32, (BLK, K), 1)
    tok = jnp.min(jnp.where(dist == rowmin, iota, K), axis=1, keepdims=True)
    tok_ref[...] = tok

    colmin_s[...] = jnp.minimum(colmin_s[...], jnp.min(dist, axis=0, keepdims=True))
    onehot = jnp.where(tok == iota, 1.0, 0.0)
    active_s[...] = jnp.maximum(active_s[...], jnp.max(onehot, axis=0, keepdims=True))
    rowsum_s[0, 0] = rowsum_s[0, 0] + jnp.sum(rowmin)

    @pl.when(i == GRID - 1)
    def _fin():
        ent = jnp.sum(jnp.where(active_s[...] > 0.0, 0.0, colmin_s[...]))
        mean_sq = rowsum_s[0, 0] / (N * D)
        total = (_EMB_W + _COMMIT_W) * mean_sq + _ENTROPY_W * ent / K
        loss_ref[...] = jnp.full((1, 1), total, jnp.float32)


def _tc_call(x, cbt, interpret=False):
    return pl.pallas_call(
        _tc_body,
        grid=(GRID,),
        in_specs=[
            pl.BlockSpec((BLK, D), lambda i: (i, 0)),
            pl.BlockSpec((D, K), lambda i: (0, 0)),
        ],
        out_specs=[
            pl.BlockSpec((BLK, 1), lambda i: (i, 0)),
            pl.BlockSpec((1, 1), lambda i: (0, 0)),
        ],
        out_shape=[
            jax.ShapeDtypeStruct((N, 1), jnp.int32),
            jax.ShapeDtypeStruct((1, 1), jnp.float32),
        ],
        scratch_shapes=[
            pltpu.VMEM((1, K), jnp.float32),
            pltpu.VMEM((1, K), jnp.float32),
            pltpu.VMEM((1, K), jnp.float32),
            pltpu.SMEM((1, 1), jnp.float32),
        ],
        interpret=interpret,
    )(x, cbt)


_SC_NC = 2
_SC_NS = 16
_NW = _SC_NC * _SC_NS
_ROWS_W = N // _NW
_IDX_CH = 128
_CH_W = _ROWS_W // _IDX_CH


def _sc_gather(tok2d, codebook):
    mesh = plsc.VectorSubcoreMesh(core_axis_name="c", subcore_axis_name="s")

    @functools.partial(
        pl.kernel,
        mesh=mesh,
        compiler_params=pltpu.CompilerParams(use_tc_tiling_on_sc=False),
        out_type=jax.ShapeDtypeStruct((N, D), jnp.float32),
        scratch_types=[
            pltpu.VMEM((_CH_W, _IDX_CH), jnp.int32),
            pltpu.VMEM((_ROWS_W, D), jnp.float32),
            pltpu.SemaphoreType.DMA,
        ],
    )
    def k(tok_hbm, cb_hbm, out_hbm, idx_v, rows_v, sem):
        wid = lax.axis_index("s") * _SC_NC + lax.axis_index("c")
        base = wid * _ROWS_W
        pltpu.sync_copy(tok_hbm.at[pl.ds(wid * _CH_W, _CH_W)], idx_v)
        for j in range(_CH_W):
            pltpu.async_copy(
                cb_hbm.at[idx_v.at[j]],
                rows_v.at[pl.ds(j * _IDX_CH, _IDX_CH)],
                sem,
            ).wait()
        pltpu.sync_copy(rows_v, out_hbm.at[pl.ds(base, _ROWS_W)])

    return k(tok2d, codebook)


def kernel(inputs, codebook):
    x = inputs.reshape(N, D)
    cbt = codebook.T
    tok, loss = _tc_call(x, cbt)
    emb = _sc_gather(tok.reshape(N // _IDX_CH, _IDX_CH), codebook)
    return emb.reshape(inputs.shape), loss[0, 0]
